# baseline (speedup 1.0000x reference)
import dataclasses
import functools

import jax
import jax.numpy as jnp
from jax import lax
from jax.experimental import pallas as pl
from jax.experimental.pallas import tpu as pltpu
from jax.experimental.pallas import tpu_sc as plsc

N = 10000
E = 320000
NC = 2
NS = 16
NW = NC * NS
CHUNK = 128
EPW = E // NW
KP = 80
HKP = KP // 2
PADE = NW * KP * CHUNK
CHUNKS_PER_CORE = (E // NC) // CHUNK
KMAX = (CHUNKS_PER_CORE + NS - 1) // NS
NACC = N + 8
RPT = 624
ZROWS = 16

_mesh = plsc.VectorSubcoreMesh(core_axis_name="c", subcore_axis_name="s")

_sc_params = pltpu.CompilerParams()
if "needs_layout_passes" in pltpu.CompilerParams.__dataclass_fields__:
    _sc_params = dataclasses.replace(_sc_params, needs_layout_passes=False)


@functools.partial(
    pl.kernel,
    mesh=_mesh,
    out_type=jax.ShapeDtypeStruct((NW, 1, 2 * N), jnp.float32),
    compiler_params=_sc_params,
    scratch_types=[
        pltpu.VMEM((EPW,), jnp.int32),
        pltpu.VMEM((EPW,), jnp.int32),
        pltpu.VMEM((2 * N,), jnp.float32),
        pltpu.SemaphoreType.DMA,
    ],
)
def _deg_kernel(src_hbm, dst_hbm, out_hbm, src_v, dst_v, h_v, sem):
    c = lax.axis_index("c")
    s = lax.axis_index("s")
    w = s * NC + c
    base = w * EPW
    pltpu.sync_copy(src_hbm.at[pl.ds(base, EPW)], src_v)
    pltpu.sync_copy(dst_hbm.at[pl.ds(base, EPW)], dst_v)

    zeros16 = jnp.zeros((16,), jnp.float32)

    @pl.loop(0, 2 * N, step=16)
    def _(i):
        h_v[pl.ds(i, 16)] = zeros16

    ones16 = jnp.ones((16,), jnp.float32)
    n_vec = jnp.full((16,), N, jnp.int32)

    @pl.loop(0, EPW, step=16)
    def _(i):
        plsc.addupdate_scatter(h_v, [src_v[pl.ds(i, 16)]], ones16)
        plsc.addupdate_scatter(h_v, [dst_v[pl.ds(i, 16)] + n_vec], ones16)

    pltpu.sync_copy(h_v, out_hbm.at[w, 0])


def _make_agg(F):
    @functools.partial(
        pl.kernel,
        mesh=_mesh,
        out_type=jax.ShapeDtypeStruct((NC, N, F), jnp.float32),
        scratch_types=[
            pltpu.VMEM((CHUNK,), jnp.int32),
            pltpu.VMEM((CHUNK,), jnp.int32),
            pltpu.VMEM((CHUNK,), jnp.int32),
            pltpu.VMEM((CHUNK,), jnp.int32),
            pltpu.VMEM((CHUNK, F), jnp.float32),
            pltpu.VMEM((CHUNK, F), jnp.float32),
            pltpu.VMEM_SHARED((NACC, F), jnp.float32),
            pltpu.SemaphoreType.DMA,
            pltpu.SemaphoreType.DMA,
            pltpu.SemaphoreType.DMA,
            pltpu.SemaphoreType.DMA,
            pltpu.SemaphoreType.DMA,
            pltpu.SemaphoreType.DMA,
        ],
    )
    def _agg(t_hbm, src_hbm, dst_hbm, out_hbm, src0_v, dst0_v, src1_v,
             dst1_v, rows0_v, rows1_v, acc_sh, sem0, sem1, sem0s, sem1s,
             semi0, semi1):
        c = lax.axis_index("c")
        s = lax.axis_index("s")
        w = s * NC + c
        row0 = s * RPT

        zeros16 = jnp.zeros((16,), jnp.float32)

        @pl.loop(0, CHUNK)
        def _(r):
            for f in range(0, F, 16):
                rows0_v[r, pl.ds(f, 16)] = zeros16

        @pl.loop(0, 512, step=CHUNK)
        def _(r):
            pltpu.sync_copy(rows0_v, acc_sh.at[pl.ds(row0 + r, CHUNK)])

        pltpu.sync_copy(rows0_v.at[pl.ds(0, RPT - 512)],
                        acc_sh.at[pl.ds(row0 + 512, RPT - 512)])

        @pl.when(s == NS - 1)
        def _():
            pltpu.sync_copy(rows0_v.at[pl.ds(0, NACC - NS * RPT)],
                            acc_sh.at[pl.ds(NS * RPT, NACC - NS * RPT)])

        plsc.subcore_barrier()

        ebase = c * (E // NC)

        @pl.loop(0, KMAX + 1, step=2)
        def _(k):
            ch0 = s + k * NS
            ch1 = s + (k + 1) * NS

            @pl.when(jnp.logical_and(k >= 2,
                                     s + (k - 2) * NS < CHUNKS_PER_CORE))
            def _():
                pltpu.make_async_copy(rows0_v, acc_sh.at[dst0_v],
                                      sem0s).wait()

            @pl.when(jnp.logical_and(k >= 2,
                                     s + (k - 1) * NS < CHUNKS_PER_CORE))
            def _():
                pltpu.make_async_copy(rows1_v, acc_sh.at[dst1_v],
                                      sem1s).wait()

            @pl.when(ch0 < CHUNKS_PER_CORE)
            def _():
                base = ebase + ch0 * CHUNK
                pltpu.async_copy(src_hbm.at[pl.ds(base, CHUNK)], src0_v,
                                 semi0)
                pltpu.async_copy(dst_hbm.at[pl.ds(base, CHUNK)], dst0_v,
                                 semi0)

            @pl.when(ch1 < CHUNKS_PER_CORE)
            def _():
                base = ebase + ch1 * CHUNK
                pltpu.async_copy(src_hbm.at[pl.ds(base, CHUNK)], src1_v,
                                 semi1)
                pltpu.async_copy(dst_hbm.at[pl.ds(base, CHUNK)], dst1_v,
                                 semi1)

            @pl.when(ch0 < CHUNKS_PER_CORE)
            def _():
                base = ebase + ch0 * CHUNK
                pltpu.make_async_copy(src_hbm.at[pl.ds(base, CHUNK)],
                                      src0_v, semi0).wait()
                pltpu.make_async_copy(dst_hbm.at[pl.ds(base, CHUNK)],
                                      dst0_v, semi0).wait()
                pltpu.async_copy(t_hbm.at[src0_v], rows0_v, sem0)

            @pl.when(ch1 < CHUNKS_PER_CORE)
            def _():
                base = ebase + ch1 * CHUNK
                pltpu.make_async_copy(src_hbm.at[pl.ds(base, CHUNK)],
                                      src1_v, semi1).wait()
                pltpu.make_async_copy(dst_hbm.at[pl.ds(base, CHUNK)],
                                      dst1_v, semi1).wait()
                pltpu.async_copy(t_hbm.at[src1_v], rows1_v, sem1)

            @pl.when(ch0 < CHUNKS_PER_CORE)
            def _():
                pltpu.make_async_copy(t_hbm.at[src0_v], rows0_v, sem0).wait()
                pltpu.async_copy(rows0_v, acc_sh.at[dst0_v], sem0s, add=True)

            @pl.when(ch1 < CHUNKS_PER_CORE)
            def _():
                pltpu.make_async_copy(t_hbm.at[src1_v], rows1_v, sem1).wait()
                pltpu.async_copy(rows1_v, acc_sh.at[dst1_v], sem1s, add=True)

        @pl.when(s + (KMAX - 1) * NS < CHUNKS_PER_CORE)
        def _():
            pltpu.make_async_copy(rows0_v, acc_sh.at[dst0_v], sem0s).wait()

        plsc.subcore_barrier()
        pltpu.sync_copy(
            acc_sh.at[pl.ds(row0, RPT)],
            out_hbm.at[c, pl.ds(row0, RPT)],
        )

        @pl.when(s == NS - 1)
        def _():
            pltpu.sync_copy(
                acc_sh.at[pl.ds(NS * RPT, ZROWS)],
                out_hbm.at[c, pl.ds(NS * RPT, ZROWS)],
            )

    return _agg


_agg128 = _make_agg(128)


def _scales_t1_body(degp_ref, x_ref, t1_ref, so_ref, si_ref):
    deg = jnp.sum(degp_ref[...], axis=0)
    so = lax.rsqrt(jnp.maximum(deg[0], 1.0))[:, None]
    so_ref[...] = so
    si_ref[...] = lax.rsqrt(jnp.maximum(deg[1], 1.0))[:, None]
    t1_ref[...] = x_ref[...] * so


def _tc_scales_t1(degp, x):
    return pl.pallas_call(
        _scales_t1_body,
        out_shape=(
            jax.ShapeDtypeStruct((N, x.shape[1]), jnp.float32),
            jax.ShapeDtypeStruct((N, 1), jnp.float32),
            jax.ShapeDtypeStruct((N, 1), jnp.float32),
        ),
    )(degp, x)


def _step2_body(p_ref, si_ref, so_ref, W1_ref, b1_ref, W2_ref, t2_ref):
    u = (p_ref[0] + p_ref[1]) * si_ref[...]
    h1 = jnp.dot(u, W1_ref[...], preferred_element_type=jnp.float32)
    h1 = jnp.maximum(h1 + b1_ref[...], 0.0)
    t2_ref[...] = jnp.dot(h1 * so_ref[...], W2_ref[...],
                          preferred_element_type=jnp.float32)


def _tc_step2(p1, si, so, W1, b1, W2):
    return pl.pallas_call(
        _step2_body,
        out_shape=jax.ShapeDtypeStruct((N, W2.shape[1]), jnp.float32),
    )(p1, si, so, W1, b1, W2)


def _step_body(p_ref, si_ref, so_ref, b_ref, W_ref, t_ref):
    h = jnp.maximum((p_ref[0] + p_ref[1]) * si_ref[...] + b_ref[...], 0.0)
    t_ref[...] = jnp.dot(h * so_ref[...], W_ref[...],
                         preferred_element_type=jnp.float32)


def _tc_step(p, si, so, b_prev, W):
    return pl.pallas_call(
        _step_body,
        out_shape=jax.ShapeDtypeStruct((N, W.shape[1]), jnp.float32),
    )(p, si, so, b_prev, W)


def _final_body(p_ref, si_ref, b_ref, o_ref):
    nc = b_ref.shape[1]
    agg = (p_ref[0, :, :nc] + p_ref[1, :, :nc]) * si_ref[...]
    o_ref[...] = jnp.maximum(agg + b_ref[...], 0.0)


def _tc_final(p, si, b):
    return pl.pallas_call(
        _final_body,
        out_shape=jax.ShapeDtypeStruct((N, b.shape[1]), jnp.float32),
    )(p, si, b)


def kernel(g, x, W1, b1, W2, b2, W3, b3, W4, b4, W5, b5):
    g = g.astype(jnp.int32)
    src, dst = g[0], g[1]
    degp = _deg_kernel(src, dst).reshape(NW, 2, N)
    t1, so, si = _tc_scales_t1(degp, x)
    p1 = _agg128(t1, src, dst)
    W3p = jnp.pad(W3, ((0, 0), (0, 128 - W3.shape[1])))
    W4p = jnp.pad(W4, ((0, 128 - W4.shape[0]), (0, 128 - W4.shape[1])))
    W5p = jnp.pad(W5, ((0, 128 - W5.shape[0]), (0, 128 - W5.shape[1])))
    b3p = jnp.pad(b3, (0, 128 - b3.shape[0])).reshape(1, -1)
    b4p = jnp.pad(b4, (0, 128 - b4.shape[0])).reshape(1, -1)
    t2 = _tc_step2(p1, si, so, W1, b1.reshape(1, -1), W2)
    p2 = _agg128(t2, src, dst)
    t3 = _tc_step(p2, si, so, b2.reshape(1, -1), W3p)
    p3 = _agg128(t3, src, dst)
    t4 = _tc_step(p3, si, so, b3p, W4p)
    p4 = _agg128(t4, src, dst)
    t5 = _tc_step(p4, si, so, b4p, W5p)
    p5 = _agg128(t5, src, dst)
    return _tc_final(p5, si, b5.reshape(1, -1))


# speedup vs baseline: 1.2037x; 1.2037x over previous
import dataclasses
import functools

import jax
import jax.numpy as jnp
from jax import lax
from jax.experimental import pallas as pl
from jax.experimental.pallas import tpu as pltpu
from jax.experimental.pallas import tpu_sc as plsc

N = 10000
E = 320000
NC = 2
NS = 16
NW = NC * NS
CHUNK = 128
EPW = E // NW
KP = 80
HKP = KP // 2
PADE = NW * KP * CHUNK
CHUNKS_PER_CORE = (E // NC) // CHUNK
KMAX = (CHUNKS_PER_CORE + NS - 1) // NS
NACC = N + 8
RPT = 624
ZROWS = 16

_mesh = plsc.VectorSubcoreMesh(core_axis_name="c", subcore_axis_name="s")

_sc_params = pltpu.CompilerParams()
if "needs_layout_passes" in pltpu.CompilerParams.__dataclass_fields__:
    _sc_params = dataclasses.replace(_sc_params, needs_layout_passes=False)


@functools.partial(
    pl.kernel,
    mesh=_mesh,
    out_type=jax.ShapeDtypeStruct((NW, 1, 2 * N), jnp.float32),
    compiler_params=_sc_params,
    scratch_types=[
        pltpu.VMEM((EPW,), jnp.int32),
        pltpu.VMEM((EPW,), jnp.int32),
        pltpu.VMEM((2 * N,), jnp.float32),
        pltpu.SemaphoreType.DMA,
    ],
)
def _deg_kernel(src_hbm, dst_hbm, out_hbm, src_v, dst_v, h_v, sem):
    c = lax.axis_index("c")
    s = lax.axis_index("s")
    w = s * NC + c
    base = w * EPW
    pltpu.sync_copy(src_hbm.at[pl.ds(base, EPW)], src_v)
    pltpu.sync_copy(dst_hbm.at[pl.ds(base, EPW)], dst_v)

    zeros16 = jnp.zeros((16,), jnp.float32)

    @pl.loop(0, 2 * N, step=16)
    def _(i):
        h_v[pl.ds(i, 16)] = zeros16

    ones16 = jnp.ones((16,), jnp.float32)
    n_vec = jnp.full((16,), N, jnp.int32)

    @pl.loop(0, EPW, step=16)
    def _(i):
        plsc.addupdate_scatter(h_v, [src_v[pl.ds(i, 16)]], ones16)
        plsc.addupdate_scatter(h_v, [dst_v[pl.ds(i, 16)] + n_vec], ones16)

    pltpu.sync_copy(h_v, out_hbm.at[w, 0])


def _make_agg(F):
    @functools.partial(
        pl.kernel,
        mesh=_mesh,
        out_type=jax.ShapeDtypeStruct((NC, N, F), jnp.float32),
        scratch_types=[
            pltpu.VMEM((CHUNK,), jnp.int32),
            pltpu.VMEM((CHUNK,), jnp.int32),
            pltpu.VMEM((CHUNK,), jnp.int32),
            pltpu.VMEM((CHUNK,), jnp.int32),
            pltpu.VMEM((CHUNK, F), jnp.float32),
            pltpu.VMEM((CHUNK, F), jnp.float32),
            pltpu.VMEM_SHARED((NACC, F), jnp.float32),
            pltpu.SemaphoreType.DMA,
            pltpu.SemaphoreType.DMA,
            pltpu.SemaphoreType.DMA,
            pltpu.SemaphoreType.DMA,
            pltpu.SemaphoreType.DMA,
            pltpu.SemaphoreType.DMA,
        ],
    )
    def _agg(t_hbm, src_hbm, dst_hbm, out_hbm, src0_v, dst0_v, src1_v,
             dst1_v, rows0_v, rows1_v, acc_sh, sem0, sem1, sem0s, sem1s,
             semi0, semi1):
        c = lax.axis_index("c")
        s = lax.axis_index("s")
        w = s * NC + c
        row0 = s * RPT

        zeros16 = jnp.zeros((16,), jnp.float32)

        @pl.loop(0, CHUNK)
        def _(r):
            for f in range(0, F, 16):
                rows0_v[r, pl.ds(f, 16)] = zeros16

        @pl.loop(0, 512, step=CHUNK)
        def _(r):
            pltpu.sync_copy(rows0_v, acc_sh.at[pl.ds(row0 + r, CHUNK)])

        pltpu.sync_copy(rows0_v.at[pl.ds(0, RPT - 512)],
                        acc_sh.at[pl.ds(row0 + 512, RPT - 512)])

        @pl.when(s == NS - 1)
        def _():
            pltpu.sync_copy(rows0_v.at[pl.ds(0, NACC - NS * RPT)],
                            acc_sh.at[pl.ds(NS * RPT, NACC - NS * RPT)])

        plsc.subcore_barrier()

        ebase = c * (E // NC)

        @pl.loop(0, KMAX + 1, step=2)
        def _(k):
            ch0 = s + k * NS
            ch1 = s + (k + 1) * NS

            @pl.when(jnp.logical_and(k >= 2,
                                     s + (k - 2) * NS < CHUNKS_PER_CORE))
            def _():
                pltpu.make_async_copy(rows0_v, acc_sh.at[dst0_v],
                                      sem0s).wait()

            @pl.when(ch0 < CHUNKS_PER_CORE)
            def _():
                base = ebase + ch0 * CHUNK
                pltpu.async_copy(src_hbm.at[pl.ds(base, CHUNK)], src0_v,
                                 semi0)
                pltpu.async_copy(dst_hbm.at[pl.ds(base, CHUNK)], dst0_v,
                                 semi0)
                pltpu.make_async_copy(src_hbm.at[pl.ds(base, CHUNK)],
                                      src0_v, semi0).wait()
                pltpu.make_async_copy(dst_hbm.at[pl.ds(base, CHUNK)],
                                      dst0_v, semi0).wait()
                pltpu.async_copy(t_hbm.at[src0_v], rows0_v, sem0)

            @pl.when(jnp.logical_and(k >= 2,
                                     s + (k - 1) * NS < CHUNKS_PER_CORE))
            def _():
                pltpu.make_async_copy(rows1_v, acc_sh.at[dst1_v],
                                      sem1s).wait()

            @pl.when(ch1 < CHUNKS_PER_CORE)
            def _():
                base = ebase + ch1 * CHUNK
                pltpu.async_copy(src_hbm.at[pl.ds(base, CHUNK)], src1_v,
                                 semi1)
                pltpu.async_copy(dst_hbm.at[pl.ds(base, CHUNK)], dst1_v,
                                 semi1)
                pltpu.make_async_copy(src_hbm.at[pl.ds(base, CHUNK)],
                                      src1_v, semi1).wait()
                pltpu.make_async_copy(dst_hbm.at[pl.ds(base, CHUNK)],
                                      dst1_v, semi1).wait()
                pltpu.async_copy(t_hbm.at[src1_v], rows1_v, sem1)

            @pl.when(ch0 < CHUNKS_PER_CORE)
            def _():
                pltpu.make_async_copy(t_hbm.at[src0_v], rows0_v, sem0).wait()
                pltpu.async_copy(rows0_v, acc_sh.at[dst0_v], sem0s, add=True)

            @pl.when(ch1 < CHUNKS_PER_CORE)
            def _():
                pltpu.make_async_copy(t_hbm.at[src1_v], rows1_v, sem1).wait()
                pltpu.async_copy(rows1_v, acc_sh.at[dst1_v], sem1s, add=True)

        @pl.when(s + (KMAX - 1) * NS < CHUNKS_PER_CORE)
        def _():
            pltpu.make_async_copy(rows0_v, acc_sh.at[dst0_v], sem0s).wait()

        plsc.subcore_barrier()
        pltpu.sync_copy(
            acc_sh.at[pl.ds(row0, RPT)],
            out_hbm.at[c, pl.ds(row0, RPT)],
        )

        @pl.when(s == NS - 1)
        def _():
            pltpu.sync_copy(
                acc_sh.at[pl.ds(NS * RPT, ZROWS)],
                out_hbm.at[c, pl.ds(NS * RPT, ZROWS)],
            )

    return _agg


_agg128 = _make_agg(128)


def _scales_t1_body(degp_ref, x_ref, t1_ref, so_ref, si_ref):
    deg = jnp.sum(degp_ref[...], axis=0)
    so = lax.rsqrt(jnp.maximum(deg[0], 1.0))[:, None]
    so_ref[...] = so
    si_ref[...] = lax.rsqrt(jnp.maximum(deg[1], 1.0))[:, None]
    t1_ref[...] = x_ref[...] * so


def _tc_scales_t1(degp, x):
    return pl.pallas_call(
        _scales_t1_body,
        out_shape=(
            jax.ShapeDtypeStruct((N, x.shape[1]), jnp.float32),
            jax.ShapeDtypeStruct((N, 1), jnp.float32),
            jax.ShapeDtypeStruct((N, 1), jnp.float32),
        ),
    )(degp, x)


def _step2_body(p_ref, si_ref, so_ref, W1_ref, b1_ref, W2_ref, t2_ref):
    u = (p_ref[0] + p_ref[1]) * si_ref[...]
    h1 = jnp.dot(u, W1_ref[...], preferred_element_type=jnp.float32)
    h1 = jnp.maximum(h1 + b1_ref[...], 0.0)
    t2_ref[...] = jnp.dot(h1 * so_ref[...], W2_ref[...],
                          preferred_element_type=jnp.float32)


def _tc_step2(p1, si, so, W1, b1, W2):
    return pl.pallas_call(
        _step2_body,
        out_shape=jax.ShapeDtypeStruct((N, W2.shape[1]), jnp.float32),
    )(p1, si, so, W1, b1, W2)


def _step_body(p_ref, si_ref, so_ref, b_ref, W_ref, t_ref):
    h = jnp.maximum((p_ref[0] + p_ref[1]) * si_ref[...] + b_ref[...], 0.0)
    t_ref[...] = jnp.dot(h * so_ref[...], W_ref[...],
                         preferred_element_type=jnp.float32)


def _tc_step(p, si, so, b_prev, W):
    return pl.pallas_call(
        _step_body,
        out_shape=jax.ShapeDtypeStruct((N, W.shape[1]), jnp.float32),
    )(p, si, so, b_prev, W)


def _final_body(p_ref, si_ref, b_ref, o_ref):
    nc = b_ref.shape[1]
    agg = (p_ref[0, :, :nc] + p_ref[1, :, :nc]) * si_ref[...]
    o_ref[...] = jnp.maximum(agg + b_ref[...], 0.0)


def _tc_final(p, si, b):
    return pl.pallas_call(
        _final_body,
        out_shape=jax.ShapeDtypeStruct((N, b.shape[1]), jnp.float32),
    )(p, si, b)


def kernel(g, x, W1, b1, W2, b2, W3, b3, W4, b4, W5, b5):
    g = g.astype(jnp.int32)
    src, dst = g[0], g[1]
    degp = _deg_kernel(src, dst).reshape(NW, 2, N)
    t1, so, si = _tc_scales_t1(degp, x)
    p1 = _agg128(t1, src, dst)
    W3p = jnp.pad(W3, ((0, 0), (0, 128 - W3.shape[1])))
    W4p = jnp.pad(W4, ((0, 128 - W4.shape[0]), (0, 128 - W4.shape[1])))
    W5p = jnp.pad(W5, ((0, 128 - W5.shape[0]), (0, 128 - W5.shape[1])))
    b3p = jnp.pad(b3, (0, 128 - b3.shape[0])).reshape(1, -1)
    b4p = jnp.pad(b4, (0, 128 - b4.shape[0])).reshape(1, -1)
    t2 = _tc_step2(p1, si, so, W1, b1.reshape(1, -1), W2)
    p2 = _agg128(t2, src, dst)
    t3 = _tc_step(p2, si, so, b2.reshape(1, -1), W3p)
    p3 = _agg128(t3, src, dst)
    t4 = _tc_step(p3, si, so, b3p, W4p)
    p4 = _agg128(t4, src, dst)
    t5 = _tc_step(p4, si, so, b4p, W5p)
    p5 = _agg128(t5, src, dst)
    return _tc_final(p5, si, b5.reshape(1, -1))


# speedup vs baseline: 1.2056x; 1.0016x over previous
import dataclasses
import functools

import jax
import jax.numpy as jnp
from jax import lax
from jax.experimental import pallas as pl
from jax.experimental.pallas import tpu as pltpu
from jax.experimental.pallas import tpu_sc as plsc

N = 10000
E = 320000
NC = 2
NS = 16
NW = NC * NS
CHUNK = 128
EPW = E // NW
CHUNKS_PER_CORE = (E // NC) // CHUNK
KMAX = (CHUNKS_PER_CORE + NS - 1) // NS
NACC = N + 8
RPT = 624
ZROWS = 16

_mesh = plsc.VectorSubcoreMesh(core_axis_name="c", subcore_axis_name="s")

_sc_params = pltpu.CompilerParams()
if "needs_layout_passes" in pltpu.CompilerParams.__dataclass_fields__:
    _sc_params = dataclasses.replace(_sc_params, needs_layout_passes=False)


@functools.partial(
    pl.kernel,
    mesh=_mesh,
    out_type=jax.ShapeDtypeStruct((NW, 1, 2 * N), jnp.float32),
    compiler_params=_sc_params,
    scratch_types=[
        pltpu.VMEM((EPW,), jnp.int32),
        pltpu.VMEM((EPW,), jnp.int32),
        pltpu.VMEM((2 * N,), jnp.float32),
        pltpu.SemaphoreType.DMA,
    ],
)
def _deg_kernel(src_hbm, dst_hbm, out_hbm, src_v, dst_v, h_v, sem):
    c = lax.axis_index("c")
    s = lax.axis_index("s")
    w = s * NC + c
    base = w * EPW
    pltpu.sync_copy(src_hbm.at[pl.ds(base, EPW)], src_v)
    pltpu.sync_copy(dst_hbm.at[pl.ds(base, EPW)], dst_v)

    zeros16 = jnp.zeros((16,), jnp.float32)

    @pl.loop(0, 2 * N, step=16)
    def _(i):
        h_v[pl.ds(i, 16)] = zeros16

    ones16 = jnp.ones((16,), jnp.float32)
    n_vec = jnp.full((16,), N, jnp.int32)

    @pl.loop(0, EPW, step=16)
    def _(i):
        plsc.addupdate_scatter(h_v, [src_v[pl.ds(i, 16)]], ones16)
        plsc.addupdate_scatter(h_v, [dst_v[pl.ds(i, 16)] + n_vec], ones16)

    pltpu.sync_copy(h_v, out_hbm.at[w, 0])


def _make_agg(F):
    @functools.partial(
        pl.kernel,
        mesh=_mesh,
        out_type=jax.ShapeDtypeStruct((NC, N, F), jnp.float32),
        scratch_types=[
            pltpu.VMEM((CHUNK,), jnp.int32),
            pltpu.VMEM((CHUNK,), jnp.int32),
            pltpu.VMEM((CHUNK,), jnp.int32),
            pltpu.VMEM((CHUNK,), jnp.int32),
            pltpu.VMEM((CHUNK, F), jnp.float32),
            pltpu.VMEM((CHUNK, F), jnp.float32),
            pltpu.VMEM_SHARED((NACC, F), jnp.float32),
            pltpu.SemaphoreType.DMA,
            pltpu.SemaphoreType.DMA,
            pltpu.SemaphoreType.DMA,
            pltpu.SemaphoreType.DMA,
            pltpu.SemaphoreType.DMA,
            pltpu.SemaphoreType.DMA,
        ],
    )
    def _agg(t_hbm, src_hbm, dst_hbm, out_hbm, src0_v, dst0_v, src1_v,
             dst1_v, rows0_v, rows1_v, acc_sh, sem0, sem1, sem0s, sem1s,
             semi0, semi1):
        c = lax.axis_index("c")
        s = lax.axis_index("s")
        w = s * NC + c
        row0 = s * RPT

        zeros16 = jnp.zeros((16,), jnp.float32)

        @pl.loop(0, CHUNK)
        def _(r):
            for f in range(0, F, 16):
                rows0_v[r, pl.ds(f, 16)] = zeros16

        @pl.loop(0, 512, step=CHUNK)
        def _(r):
            pltpu.sync_copy(rows0_v, acc_sh.at[pl.ds(row0 + r, CHUNK)])

        pltpu.sync_copy(rows0_v.at[pl.ds(0, RPT - 512)],
                        acc_sh.at[pl.ds(row0 + 512, RPT - 512)])

        @pl.when(s == NS - 1)
        def _():
            pltpu.sync_copy(rows0_v.at[pl.ds(0, NACC - NS * RPT)],
                            acc_sh.at[pl.ds(NS * RPT, NACC - NS * RPT)])

        plsc.subcore_barrier()

        ebase = c * (E // NC)

        @pl.loop(0, KMAX + 1, step=2)
        def _(k):
            ch0 = s + k * NS
            ch1 = s + (k + 1) * NS

            @pl.when(jnp.logical_and(k >= 2,
                                     s + (k - 2) * NS < CHUNKS_PER_CORE))
            def _():
                pltpu.make_async_copy(rows0_v, acc_sh.at[dst0_v],
                                      sem0s).wait()

            @pl.when(ch0 < CHUNKS_PER_CORE)
            def _():
                base = ebase + ch0 * CHUNK
                pltpu.async_copy(src_hbm.at[pl.ds(base, CHUNK)], src0_v,
                                 semi0)
                pltpu.async_copy(dst_hbm.at[pl.ds(base, CHUNK)], dst0_v,
                                 semi0)
                pltpu.make_async_copy(src_hbm.at[pl.ds(base, CHUNK)],
                                      src0_v, semi0).wait()
                pltpu.make_async_copy(dst_hbm.at[pl.ds(base, CHUNK)],
                                      dst0_v, semi0).wait()
                pltpu.async_copy(t_hbm.at[src0_v], rows0_v, sem0)

            @pl.when(jnp.logical_and(k >= 2,
                                     s + (k - 1) * NS < CHUNKS_PER_CORE))
            def _():
                pltpu.make_async_copy(rows1_v, acc_sh.at[dst1_v],
                                      sem1s).wait()

            @pl.when(ch1 < CHUNKS_PER_CORE)
            def _():
                base = ebase + ch1 * CHUNK
                pltpu.async_copy(src_hbm.at[pl.ds(base, CHUNK)], src1_v,
                                 semi1)
                pltpu.async_copy(dst_hbm.at[pl.ds(base, CHUNK)], dst1_v,
                                 semi1)
                pltpu.make_async_copy(src_hbm.at[pl.ds(base, CHUNK)],
                                      src1_v, semi1).wait()
                pltpu.make_async_copy(dst_hbm.at[pl.ds(base, CHUNK)],
                                      dst1_v, semi1).wait()
                pltpu.async_copy(t_hbm.at[src1_v], rows1_v, sem1)

            @pl.when(ch0 < CHUNKS_PER_CORE)
            def _():
                pltpu.make_async_copy(t_hbm.at[src0_v], rows0_v, sem0).wait()
                pltpu.async_copy(rows0_v, acc_sh.at[dst0_v], sem0s, add=True)

            @pl.when(ch1 < CHUNKS_PER_CORE)
            def _():
                pltpu.make_async_copy(t_hbm.at[src1_v], rows1_v, sem1).wait()
                pltpu.async_copy(rows1_v, acc_sh.at[dst1_v], sem1s, add=True)

        @pl.when(s + (KMAX - 1) * NS < CHUNKS_PER_CORE)
        def _():
            pltpu.make_async_copy(rows0_v, acc_sh.at[dst0_v], sem0s).wait()

        plsc.subcore_barrier()
        pltpu.sync_copy(
            acc_sh.at[pl.ds(row0, RPT)],
            out_hbm.at[c, pl.ds(row0, RPT)],
        )

        @pl.when(s == NS - 1)
        def _():
            pltpu.sync_copy(
                acc_sh.at[pl.ds(NS * RPT, ZROWS)],
                out_hbm.at[c, pl.ds(NS * RPT, ZROWS)],
            )

    return _agg


_agg128 = _make_agg(128)


def _scales_t1_body(degp_ref, x_ref, t1_ref, so_ref, si_ref):
    deg = jnp.sum(degp_ref[...], axis=0)
    so = lax.rsqrt(jnp.maximum(deg[0], 1.0))[:, None]
    so_ref[...] = so
    si_ref[...] = lax.rsqrt(jnp.maximum(deg[1], 1.0))[:, None]
    t1_ref[...] = x_ref[...] * so


def _tc_scales_t1(degp, x):
    return pl.pallas_call(
        _scales_t1_body,
        out_shape=(
            jax.ShapeDtypeStruct((N, x.shape[1]), jnp.float32),
            jax.ShapeDtypeStruct((N, 1), jnp.float32),
            jax.ShapeDtypeStruct((N, 1), jnp.float32),
        ),
    )(degp, x)


def _step2_body(p_ref, si_ref, so_ref, W1_ref, b1_ref, W2_ref, t2_ref):
    u = (p_ref[0] + p_ref[1]) * si_ref[...]
    h1 = jnp.dot(u, W1_ref[...], preferred_element_type=jnp.float32)
    h1 = jnp.maximum(h1 + b1_ref[...], 0.0)
    t2_ref[...] = jnp.dot(h1 * so_ref[...], W2_ref[...],
                          preferred_element_type=jnp.float32)


def _tc_step2(p1, si, so, W1, b1, W2):
    return pl.pallas_call(
        _step2_body,
        out_shape=jax.ShapeDtypeStruct((N, W2.shape[1]), jnp.float32),
    )(p1, si, so, W1, b1, W2)


def _step_body(p_ref, si_ref, so_ref, b_ref, W_ref, t_ref):
    h = jnp.maximum((p_ref[0] + p_ref[1]) * si_ref[...] + b_ref[...], 0.0)
    t_ref[...] = jnp.dot(h * so_ref[...], W_ref[...],
                         preferred_element_type=jnp.float32)


def _tc_step(p, si, so, b_prev, W):
    return pl.pallas_call(
        _step_body,
        out_shape=jax.ShapeDtypeStruct((N, W.shape[1]), jnp.float32),
    )(p, si, so, b_prev, W)


def _final_body(p_ref, si_ref, b_ref, o_ref):
    nc = b_ref.shape[1]
    agg = (p_ref[0, :, :nc] + p_ref[1, :, :nc]) * si_ref[...]
    o_ref[...] = jnp.maximum(agg + b_ref[...], 0.0)


def _tc_final(p, si, b):
    return pl.pallas_call(
        _final_body,
        out_shape=jax.ShapeDtypeStruct((N, b.shape[1]), jnp.float32),
    )(p, si, b)


def kernel(g, x, W1, b1, W2, b2, W3, b3, W4, b4, W5, b5):
    g = g.astype(jnp.int32)
    src, dst = g[0], g[1]
    degp = _deg_kernel(src, dst).reshape(NW, 2, N)
    t1, so, si = _tc_scales_t1(degp, x)
    p1 = _agg128(t1, src, dst)
    W3p = jnp.pad(W3, ((0, 0), (0, 128 - W3.shape[1])))
    W4p = jnp.pad(W4, ((0, 128 - W4.shape[0]), (0, 128 - W4.shape[1])))
    W5p = jnp.pad(W5, ((0, 128 - W5.shape[0]), (0, 128 - W5.shape[1])))
    b3p = jnp.pad(b3, (0, 128 - b3.shape[0])).reshape(1, -1)
    b4p = jnp.pad(b4, (0, 128 - b4.shape[0])).reshape(1, -1)
    t2 = _tc_step2(p1, si, so, W1, b1.reshape(1, -1), W2)
    p2 = _agg128(t2, src, dst)
    t3 = _tc_step(p2, si, so, b2.reshape(1, -1), W3p)
    p3 = _agg128(t3, src, dst)
    t4 = _tc_step(p3, si, so, b3p, W4p)
    p4 = _agg128(t4, src, dst)
    t5 = _tc_step(p4, si, so, b4p, W5p)
    p5 = _agg128(t5, src, dst)
    return _tc_final(p5, si, b5.reshape(1, -1))
